# Initial kernel scaffold; baseline (speedup 1.0000x reference)
import jax
import jax.numpy as jnp
from jax.experimental import pallas as pl


def kernel(x, edge_index, batch, W1, b1, W2, b2):
    raise NotImplementedError("write your pallas kernel here")


# speedup vs baseline: 14.5537x; 14.5537x over previous
import functools

import jax
import jax.numpy as jnp
from jax import lax
from jax.experimental import pallas as pl
from jax.experimental.pallas import tpu as pltpu
from jax.experimental.pallas import tpu_sc as plsc

N = 10000
E = 320000
G = 64
DH = 128

NC = 2
NS = 16
NW = NC * NS
EPW = E // NW
K = 80
CH = EPW // K
NP = 10240
RPS = NP // NS
ZR = 128

_MESH = plsc.VectorSubcoreMesh(
    core_axis_name="c", subcore_axis_name="s", num_cores=NC, num_subcores=NS)


def _deg_body(dstm, ones_hbm, zeros_hbm, out, acc, ones_v, idx_v):
    c = lax.axis_index("c")
    s = lax.axis_index("s")
    wid = s * NC + c
    pltpu.sync_copy(ones_hbm, ones_v)
    for r in range(RPS // ZR):
        pltpu.sync_copy(zeros_hbm, acc.at[pl.ds(s * RPS + r * ZR, ZR)])
    pltpu.sync_copy(dstm.at[wid], idx_v)
    plsc.subcore_barrier()

    def body(j, carry):
        pltpu.sync_copy(ones_v, acc.at[idx_v.at[j]], add=True)
        return carry

    lax.fori_loop(0, CH, body, 0)
    plsc.subcore_barrier()
    pltpu.sync_copy(acc.at[pl.ds(s * RPS, RPS)],
                    out.at[c, pl.ds(s * RPS, RPS)])


_deg_call = pl.kernel(
    _deg_body,
    out_type=jax.ShapeDtypeStruct((NC, NP, DH), jnp.float32),
    mesh=_MESH,
    scratch_types=[
        pltpu.VMEM_SHARED((NP, DH), jnp.float32),
        pltpu.VMEM((K, DH), jnp.float32),
        pltpu.VMEM((CH, K), jnp.int32),
    ],
)


def _msg_body(hs, srcm, dstm, zeros_hbm, out, acc, src_v, dst_v, rows_v, sem):
    c = lax.axis_index("c")
    s = lax.axis_index("s")
    wid = s * NC + c
    for r in range(RPS // ZR):
        pltpu.sync_copy(zeros_hbm, acc.at[pl.ds(s * RPS + r * ZR, ZR)])
    pltpu.sync_copy(srcm.at[wid], src_v)
    pltpu.sync_copy(dstm.at[wid], dst_v)
    plsc.subcore_barrier()

    def body(j, carry):
        pltpu.async_copy(hs.at[src_v.at[j]], rows_v, sem).wait()
        pltpu.sync_copy(rows_v, acc.at[dst_v.at[j]], add=True)
        return carry

    lax.fori_loop(0, CH, body, 0)
    plsc.subcore_barrier()
    pltpu.sync_copy(acc.at[pl.ds(s * RPS, RPS)],
                    out.at[c, pl.ds(s * RPS, RPS)])


_msg_call = pl.kernel(
    _msg_body,
    out_type=jax.ShapeDtypeStruct((NC, NP, DH), jnp.float32),
    mesh=_MESH,
    scratch_types=[
        pltpu.VMEM_SHARED((NP, DH), jnp.float32),
        pltpu.VMEM((CH, K), jnp.int32),
        pltpu.VMEM((CH, K), jnp.int32),
        pltpu.VMEM((K, DH), jnp.float32),
        pltpu.SemaphoreType.DMA,
    ],
)


_RB = 400


def _mm1_body(x_ref, w_ref, d0_ref, d1_ref, o_ref):
    dis = lax.rsqrt(d0_ref[...] + d1_ref[...] + 1.0)
    o_ref[...] = jnp.dot(x_ref[...], w_ref[...],
                         preferred_element_type=jnp.float32) * dis


def _layer2_body(p0_ref, p1_ref, hs1_ref, d0_ref, d1_ref, b1_ref, w2_ref,
                 o_ref):
    dis = lax.rsqrt(d0_ref[...] + d1_ref[...] + 1.0)
    agg = (p0_ref[...] + p1_ref[...] + hs1_ref[...]) * dis + b1_ref[...]
    h = jnp.maximum(agg, 0.0)
    o_ref[...] = jnp.dot(h, w2_ref[...],
                         preferred_element_type=jnp.float32) * dis


def _pool_body(p0_ref, p1_ref, hs2_ref, d0_ref, d1_ref, b2_ref, batch_ref,
               o_ref, sums, cnts):
    i = pl.program_id(0)

    @pl.when(i == 0)
    def _init():
        sums[...] = jnp.zeros_like(sums)
        cnts[...] = jnp.zeros_like(cnts)

    dis = lax.rsqrt(d0_ref[...] + d1_ref[...] + 1.0)
    p2 = (p0_ref[...] + p1_ref[...] + hs2_ref[...]) * dis + b2_ref[...]
    gids = lax.broadcasted_iota(jnp.int32, (_RB, G), 1)
    onehot = (batch_ref[...] == gids).astype(jnp.float32)
    sums[...] += lax.dot_general(onehot, p2, (((0,), (0,)), ((), ())),
                                 preferred_element_type=jnp.float32)
    cnts[...] += lax.dot_general(onehot, jnp.ones((_RB, 1), jnp.float32),
                                 (((0,), (0,)), ((), ())),
                                 preferred_element_type=jnp.float32)

    @pl.when(i == pl.num_programs(0) - 1)
    def _fin():
        o_ref[...] = sums[...] / jnp.maximum(cnts[...], 1.0)


def _row_spec(w):
    return pl.BlockSpec((_RB, w), lambda i: (i, 0))


def _full_spec(shape):
    return pl.BlockSpec(shape, lambda i: (0,) * len(shape))


_mm1_call = pl.pallas_call(
    _mm1_body,
    grid=(N // _RB,),
    in_specs=[_row_spec(768), _full_spec((768, DH)),
              _row_spec(1), _row_spec(1)],
    out_specs=_row_spec(DH),
    out_shape=jax.ShapeDtypeStruct((N, DH), jnp.float32),
)

_layer2_call = pl.pallas_call(
    _layer2_body,
    grid=(N // _RB,),
    in_specs=[_row_spec(DH), _row_spec(DH), _row_spec(DH),
              _row_spec(1), _row_spec(1),
              _full_spec((1, DH)), _full_spec((DH, DH))],
    out_specs=_row_spec(DH),
    out_shape=jax.ShapeDtypeStruct((N, DH), jnp.float32),
)

_pool_call = pl.pallas_call(
    _pool_body,
    grid=(N // _RB,),
    in_specs=[_row_spec(DH), _row_spec(DH), _row_spec(DH),
              _row_spec(1), _row_spec(1),
              _full_spec((1, DH)), _row_spec(1)],
    out_specs=_full_spec((G, DH)),
    out_shape=jax.ShapeDtypeStruct((G, DH), jnp.float32),
    scratch_shapes=[pltpu.VMEM((G, DH), jnp.float32),
                    pltpu.VMEM((G, 1), jnp.float32)],
)


def kernel(x, edge_index, batch, W1, b1, W2, b2):
    srcm = edge_index[0].astype(jnp.int32).reshape(NW, CH, K)
    dstm = edge_index[1].astype(jnp.int32).reshape(NW, CH, K)
    ones128 = jnp.ones((K, DH), jnp.float32)
    zeros128 = jnp.zeros((ZR, DH), jnp.float32)
    batch2 = batch.astype(jnp.int32).reshape(N, 1)
    b1r = b1.reshape(1, DH)
    b2r = b2.reshape(1, DH)

    degp = _deg_call(dstm, ones128, zeros128)
    d0 = degp[0, :N, 0:1]
    d1 = degp[1, :N, 0:1]

    hs1 = _mm1_call(x, W1, d0, d1)
    m1 = _msg_call(hs1, srcm, dstm, zeros128)
    hs2 = _layer2_call(m1[0, :N], m1[1, :N], hs1, d0, d1, b1r, W2)
    m2 = _msg_call(hs2, srcm, dstm, zeros128)
    return _pool_call(m2[0, :N], m2[1, :N], hs2, d0, d1, b2r, batch2)


# speedup vs baseline: 19.2871x; 1.3252x over previous
import functools

import jax
import jax.numpy as jnp
from jax import lax
from jax.experimental import pallas as pl
from jax.experimental.pallas import tpu as pltpu
from jax.experimental.pallas import tpu_sc as plsc

N = 10000
E = 320000
G = 64
DH = 128

NC = 2
NS = 16
NW = NC * NS
EPW = E // NW
K = 80
CH = EPW // K
NP = 10240
RPS = NP // NS
ZR = 128
IB = 25
NB = CH // IB

_MESH = plsc.VectorSubcoreMesh(
    core_axis_name="c", subcore_axis_name="s", num_cores=NC, num_subcores=NS)


def _deg_body(dstm, ones_hbm, zeros_hbm, out, acc, ones_v, idx_v):
    c = lax.axis_index("c")
    s = lax.axis_index("s")
    wid = s * NC + c
    pltpu.sync_copy(ones_hbm, ones_v)
    for r in range(RPS // ZR):
        pltpu.sync_copy(zeros_hbm, acc.at[pl.ds(s * RPS + r * ZR, ZR)])
    plsc.subcore_barrier()

    for b in range(NB):
        pltpu.sync_copy(dstm.at[wid, b], idx_v)

        def body(j, carry):
            pltpu.sync_copy(ones_v, acc.at[idx_v.at[j]], add=True)
            return carry

        lax.fori_loop(0, IB, body, 0)
    plsc.subcore_barrier()
    pltpu.sync_copy(acc.at[pl.ds(s * RPS, RPS)],
                    out.at[c, pl.ds(s * RPS, RPS)])


_deg_call = pl.kernel(
    _deg_body,
    out_type=jax.ShapeDtypeStruct((NC, NP, DH), jnp.float32),
    mesh=_MESH,
    scratch_types=[
        pltpu.VMEM_SHARED((NP, DH), jnp.float32),
        pltpu.VMEM((K, DH), jnp.float32),
        pltpu.VMEM((IB, K), jnp.int32),
    ],
)


def _msg_body(hs, srcm, dstm, zeros_hbm, out, acc, src_v, dst_v,
              rows0, rows1, sem0, sem1):
    c = lax.axis_index("c")
    s = lax.axis_index("s")
    wid = s * NC + c
    for r in range(RPS // ZR):
        pltpu.sync_copy(zeros_hbm, acc.at[pl.ds(s * RPS + r * ZR, ZR)])
    plsc.subcore_barrier()

    for b in range(NB):
        pltpu.sync_copy(srcm.at[wid, b], src_v)
        pltpu.sync_copy(dstm.at[wid, b], dst_v)
        pltpu.async_copy(hs.at[src_v.at[0]], rows0, sem0)

        def pair(p, carry):
            j = 2 * p
            pltpu.async_copy(hs.at[src_v.at[j + 1]], rows1, sem1)
            pltpu.make_async_copy(hs.at[src_v.at[j]], rows0, sem0).wait()
            pltpu.sync_copy(rows0, acc.at[dst_v.at[j]], add=True)
            pltpu.async_copy(hs.at[src_v.at[j + 2]], rows0, sem0)
            pltpu.make_async_copy(hs.at[src_v.at[j + 1]], rows1, sem1).wait()
            pltpu.sync_copy(rows1, acc.at[dst_v.at[j + 1]], add=True)
            return carry

        lax.fori_loop(0, (IB - 1) // 2, pair, 0)
        pltpu.make_async_copy(hs.at[src_v.at[IB - 1]], rows0, sem0).wait()
        pltpu.sync_copy(rows0, acc.at[dst_v.at[IB - 1]], add=True)
    plsc.subcore_barrier()
    pltpu.sync_copy(acc.at[pl.ds(s * RPS, RPS)],
                    out.at[c, pl.ds(s * RPS, RPS)])


_msg_call = pl.kernel(
    _msg_body,
    out_type=jax.ShapeDtypeStruct((NC, NP, DH), jnp.float32),
    mesh=_MESH,
    scratch_types=[
        pltpu.VMEM_SHARED((NP, DH), jnp.float32),
        pltpu.VMEM((IB, K), jnp.int32),
        pltpu.VMEM((IB, K), jnp.int32),
        pltpu.VMEM((K, DH), jnp.float32),
        pltpu.VMEM((K, DH), jnp.float32),
        pltpu.SemaphoreType.DMA,
        pltpu.SemaphoreType.DMA,
    ],
)


_RB = 400


def _mm1_body(x_ref, w_ref, d0_ref, d1_ref, o_ref):
    dis = lax.rsqrt(d0_ref[...] + d1_ref[...] + 1.0)
    o_ref[...] = jnp.dot(x_ref[...], w_ref[...],
                         preferred_element_type=jnp.float32) * dis


def _layer2_body(p0_ref, p1_ref, hs1_ref, d0_ref, d1_ref, b1_ref, w2_ref,
                 o_ref):
    dis = lax.rsqrt(d0_ref[...] + d1_ref[...] + 1.0)
    agg = (p0_ref[...] + p1_ref[...] + hs1_ref[...]) * dis + b1_ref[...]
    h = jnp.maximum(agg, 0.0)
    o_ref[...] = jnp.dot(h, w2_ref[...],
                         preferred_element_type=jnp.float32) * dis


def _pool_body(p0_ref, p1_ref, hs2_ref, d0_ref, d1_ref, b2_ref, batch_ref,
               o_ref, sums, cnts):
    i = pl.program_id(0)

    @pl.when(i == 0)
    def _init():
        sums[...] = jnp.zeros_like(sums)
        cnts[...] = jnp.zeros_like(cnts)

    dis = lax.rsqrt(d0_ref[...] + d1_ref[...] + 1.0)
    p2 = (p0_ref[...] + p1_ref[...] + hs2_ref[...]) * dis + b2_ref[...]
    gids = lax.broadcasted_iota(jnp.int32, (_RB, G), 1)
    onehot = (batch_ref[...] == gids).astype(jnp.float32)
    sums[...] += lax.dot_general(onehot, p2, (((0,), (0,)), ((), ())),
                                 preferred_element_type=jnp.float32)
    cnts[...] += lax.dot_general(onehot, jnp.ones((_RB, 1), jnp.float32),
                                 (((0,), (0,)), ((), ())),
                                 preferred_element_type=jnp.float32)

    @pl.when(i == pl.num_programs(0) - 1)
    def _fin():
        o_ref[...] = sums[...] / jnp.maximum(cnts[...], 1.0)


def _row_spec(w):
    return pl.BlockSpec((_RB, w), lambda i: (i, 0))


def _full_spec(shape):
    return pl.BlockSpec(shape, lambda i: (0,) * len(shape))


_mm1_call = pl.pallas_call(
    _mm1_body,
    grid=(N // _RB,),
    in_specs=[_row_spec(768), _full_spec((768, DH)),
              _row_spec(1), _row_spec(1)],
    out_specs=_row_spec(DH),
    out_shape=jax.ShapeDtypeStruct((N, DH), jnp.float32),
)

_layer2_call = pl.pallas_call(
    _layer2_body,
    grid=(N // _RB,),
    in_specs=[_row_spec(DH), _row_spec(DH), _row_spec(DH),
              _row_spec(1), _row_spec(1),
              _full_spec((1, DH)), _full_spec((DH, DH))],
    out_specs=_row_spec(DH),
    out_shape=jax.ShapeDtypeStruct((N, DH), jnp.float32),
)

_pool_call = pl.pallas_call(
    _pool_body,
    grid=(N // _RB,),
    in_specs=[_row_spec(DH), _row_spec(DH), _row_spec(DH),
              _row_spec(1), _row_spec(1),
              _full_spec((1, DH)), _row_spec(1)],
    out_specs=_full_spec((G, DH)),
    out_shape=jax.ShapeDtypeStruct((G, DH), jnp.float32),
    scratch_shapes=[pltpu.VMEM((G, DH), jnp.float32),
                    pltpu.VMEM((G, 1), jnp.float32)],
)


def kernel(x, edge_index, batch, W1, b1, W2, b2):
    srcm = edge_index[0].astype(jnp.int32).reshape(NW, NB, IB, K)
    dstm = edge_index[1].astype(jnp.int32).reshape(NW, NB, IB, K)
    ones128 = jnp.ones((K, DH), jnp.float32)
    zeros128 = jnp.zeros((ZR, DH), jnp.float32)
    batch2 = batch.astype(jnp.int32).reshape(N, 1)
    b1r = b1.reshape(1, DH)
    b2r = b2.reshape(1, DH)

    degp = _deg_call(dstm, ones128, zeros128)
    d0 = degp[0, :N, 0:1]
    d1 = degp[1, :N, 0:1]

    hs1 = _mm1_call(x, W1, d0, d1)
    m1 = _msg_call(hs1, srcm, dstm, zeros128)
    hs2 = _layer2_call(m1[0, :N], m1[1, :N], hs1, d0, d1, b1r, W2)
    m2 = _msg_call(hs2, srcm, dstm, zeros128)
    return _pool_call(m2[0, :N], m2[1, :N], hs2, d0, d1, b2r, batch2)


# speedup vs baseline: 21.1680x; 1.0975x over previous
import functools

import jax
import jax.numpy as jnp
from jax import lax
from jax.experimental import pallas as pl
from jax.experimental.pallas import tpu as pltpu
from jax.experimental.pallas import tpu_sc as plsc

N = 10000
E = 320000
G = 64
DH = 128

NC = 2
NS = 16
NW = NC * NS
EPW = E // NW
K = 80
CH = EPW // K
NP = 10240
RPS = NP // NS
ZR = 128
IB = 25
NB = CH // IB

_MESH = plsc.VectorSubcoreMesh(
    core_axis_name="c", subcore_axis_name="s", num_cores=NC, num_subcores=NS)


def _deg_body(dstm, ones_hbm, zeros_hbm, out, acc, ones_v, idx_v):
    c = lax.axis_index("c")
    s = lax.axis_index("s")
    wid = s * NC + c
    pltpu.sync_copy(ones_hbm, ones_v)
    for r in range(RPS // ZR):
        pltpu.sync_copy(zeros_hbm, acc.at[pl.ds(s * RPS + r * ZR, ZR)])
    plsc.subcore_barrier()

    for b in range(NB):
        pltpu.sync_copy(dstm.at[wid, b], idx_v)

        def body(j, carry):
            pltpu.sync_copy(ones_v, acc.at[idx_v.at[j]], add=True)
            return carry

        lax.fori_loop(0, IB, body, 0)
    plsc.subcore_barrier()
    pltpu.sync_copy(acc.at[pl.ds(s * RPS, RPS)],
                    out.at[c, pl.ds(s * RPS, RPS)])


_deg_call = pl.kernel(
    _deg_body,
    out_type=jax.ShapeDtypeStruct((NC, NP, DH), jnp.float32),
    mesh=_MESH,
    scratch_types=[
        pltpu.VMEM_SHARED((NP, DH), jnp.float32),
        pltpu.VMEM((K, DH), jnp.float32),
        pltpu.VMEM((IB, K), jnp.int32),
    ],
)


def _msg_body(hs, srcm, dstm, zeros_hbm, out, acc, src_v, dst_v,
              rows0, rows1, rows2, sem0, sem1, sem2):
    c = lax.axis_index("c")
    s = lax.axis_index("s")
    wid = s * NC + c
    for r in range(RPS // ZR):
        pltpu.sync_copy(zeros_hbm, acc.at[pl.ds(s * RPS + r * ZR, ZR)])
    plsc.subcore_barrier()

    bufs = (rows0, rows1, rows2)
    sems = (sem0, sem1, sem2)

    def gather(j, b):
        pltpu.async_copy(hs.at[src_v.at[j]], bufs[b], sems[b])

    def wait_scat(j, b):
        pltpu.make_async_copy(hs.at[src_v.at[j]], bufs[b], sems[b]).wait()
        pltpu.sync_copy(bufs[b], acc.at[dst_v.at[j]], add=True)

    for blk in range(NB):
        pltpu.sync_copy(srcm.at[wid, blk], src_v)
        pltpu.sync_copy(dstm.at[wid, blk], dst_v)
        gather(0, 0)
        gather(1, 1)

        def tri(p, carry):
            j = 3 * p
            gather(j + 2, 2)
            wait_scat(j, 0)
            gather(j + 3, 0)
            wait_scat(j + 1, 1)
            gather(j + 4, 1)
            wait_scat(j + 2, 2)
            return carry

        lax.fori_loop(0, (IB - 4) // 3, tri, 0)
        gather(IB - 2, 2)
        wait_scat(IB - 4, 0)
        gather(IB - 1, 0)
        wait_scat(IB - 3, 1)
        wait_scat(IB - 2, 2)
        wait_scat(IB - 1, 0)
    plsc.subcore_barrier()
    pltpu.sync_copy(acc.at[pl.ds(s * RPS, RPS)],
                    out.at[c, pl.ds(s * RPS, RPS)])


_msg_call = pl.kernel(
    _msg_body,
    out_type=jax.ShapeDtypeStruct((NC, NP, DH), jnp.float32),
    mesh=_MESH,
    scratch_types=[
        pltpu.VMEM_SHARED((NP, DH), jnp.float32),
        pltpu.VMEM((IB, K), jnp.int32),
        pltpu.VMEM((IB, K), jnp.int32),
        pltpu.VMEM((K, DH), jnp.float32),
        pltpu.VMEM((K, DH), jnp.float32),
        pltpu.VMEM((K, DH), jnp.float32),
        pltpu.SemaphoreType.DMA,
        pltpu.SemaphoreType.DMA,
        pltpu.SemaphoreType.DMA,
    ],
)


_RB = 400


def _mm1_body(x_ref, w_ref, d0_ref, d1_ref, o_ref):
    dis = lax.rsqrt(d0_ref[0] + d1_ref[0] + 1.0)
    o_ref[...] = jnp.dot(x_ref[...], w_ref[...],
                         preferred_element_type=jnp.float32) * dis


def _layer2_body(p0_ref, p1_ref, hs1_ref, d0_ref, d1_ref, b1_ref, w2_ref,
                 o_ref):
    dis = lax.rsqrt(d0_ref[0] + d1_ref[0] + 1.0)
    agg = (p0_ref[0] + p1_ref[0] + hs1_ref[...]) * dis + b1_ref[...]
    h = jnp.maximum(agg, 0.0)
    o_ref[...] = jnp.dot(h, w2_ref[...],
                         preferred_element_type=jnp.float32) * dis


def _pool_body(p0_ref, p1_ref, hs2_ref, d0_ref, d1_ref, b2_ref, batch_ref,
               o_ref, sums, cnts):
    i = pl.program_id(0)

    @pl.when(i == 0)
    def _init():
        sums[...] = jnp.zeros_like(sums)
        cnts[...] = jnp.zeros_like(cnts)

    dis = lax.rsqrt(d0_ref[0] + d1_ref[0] + 1.0)
    p2 = (p0_ref[0] + p1_ref[0] + hs2_ref[...]) * dis + b2_ref[...]
    gids = lax.broadcasted_iota(jnp.int32, (_RB, G), 1)
    onehot = (batch_ref[...] == gids).astype(jnp.float32)
    sums[...] += lax.dot_general(onehot, p2, (((0,), (0,)), ((), ())),
                                 preferred_element_type=jnp.float32)
    cnts[...] += lax.dot_general(onehot, jnp.ones((_RB, 1), jnp.float32),
                                 (((0,), (0,)), ((), ())),
                                 preferred_element_type=jnp.float32)

    @pl.when(i == pl.num_programs(0) - 1)
    def _fin():
        o_ref[...] = sums[...] / jnp.maximum(cnts[...], 1.0)


def _row_spec(w):
    return pl.BlockSpec((_RB, w), lambda i: (i, 0))


def _part_spec(part, w):
    return pl.BlockSpec((1, _RB, w), lambda i, _p=part: (_p, i, 0))


def _full_spec(shape):
    return pl.BlockSpec(shape, lambda i: (0,) * len(shape))


_mm1_call = pl.pallas_call(
    _mm1_body,
    grid=(N // _RB,),
    in_specs=[_row_spec(768), _full_spec((768, DH)),
              _part_spec(0, 1), _part_spec(1, 1)],
    out_specs=_row_spec(DH),
    out_shape=jax.ShapeDtypeStruct((N, DH), jnp.float32),
)

_layer2_call = pl.pallas_call(
    _layer2_body,
    grid=(N // _RB,),
    in_specs=[_part_spec(0, DH), _part_spec(1, DH), _row_spec(DH),
              _part_spec(0, 1), _part_spec(1, 1),
              _full_spec((1, DH)), _full_spec((DH, DH))],
    out_specs=_row_spec(DH),
    out_shape=jax.ShapeDtypeStruct((N, DH), jnp.float32),
)

_pool_call = pl.pallas_call(
    _pool_body,
    grid=(N // _RB,),
    in_specs=[_part_spec(0, DH), _part_spec(1, DH), _row_spec(DH),
              _part_spec(0, 1), _part_spec(1, 1),
              _full_spec((1, DH)), _row_spec(1)],
    out_specs=_full_spec((G, DH)),
    out_shape=jax.ShapeDtypeStruct((G, DH), jnp.float32),
    scratch_shapes=[pltpu.VMEM((G, DH), jnp.float32),
                    pltpu.VMEM((G, 1), jnp.float32)],
)


def kernel(x, edge_index, batch, W1, b1, W2, b2):
    srcm = edge_index[0].astype(jnp.int32).reshape(NW, NB, IB, K)
    dstm = edge_index[1].astype(jnp.int32).reshape(NW, NB, IB, K)
    ones128 = jnp.ones((K, DH), jnp.float32)
    zeros128 = jnp.zeros((ZR, DH), jnp.float32)
    batch2 = batch.astype(jnp.int32).reshape(N, 1)
    b1r = b1.reshape(1, DH)
    b2r = b2.reshape(1, DH)

    degp = _deg_call(dstm, ones128, zeros128)
    degc = degp[:, :, 0:1]
    hs1 = _mm1_call(x, W1, degc, degc)
    m1 = _msg_call(hs1, srcm, dstm, zeros128)
    hs2 = _layer2_call(m1, m1, hs1, degc, degc, b1r, W2)
    m2 = _msg_call(hs2, srcm, dstm, zeros128)
    return _pool_call(m2, m2, hs2, degc, degc, b2r, batch2)


# speedup vs baseline: 21.3194x; 1.0072x over previous
import functools

import jax
import jax.numpy as jnp
from jax import lax
from jax.experimental import pallas as pl
from jax.experimental.pallas import tpu as pltpu
from jax.experimental.pallas import tpu_sc as plsc

N = 10000
E = 320000
G = 64
DH = 128

NC = 2
NS = 16
NW = NC * NS
EPW = E // NW
K = 80
CH = EPW // K
NP = 10240
RPS = NP // NS
ZR = 128
IB = 25
NB = CH // IB

_MESH = plsc.VectorSubcoreMesh(
    core_axis_name="c", subcore_axis_name="s", num_cores=NC, num_subcores=NS)


def _deg_body(dstm, ones_hbm, zeros_hbm, out, acc, ones_v, idx_v):
    c = lax.axis_index("c")
    s = lax.axis_index("s")
    wid = s * NC + c
    pltpu.sync_copy(ones_hbm, ones_v)
    for r in range(RPS // ZR):
        pltpu.sync_copy(zeros_hbm, acc.at[pl.ds(s * RPS + r * ZR, ZR)])
    plsc.subcore_barrier()

    for b in range(NB):
        pltpu.sync_copy(dstm.at[wid, b], idx_v)

        def body(j, carry):
            pltpu.sync_copy(ones_v, acc.at[idx_v.at[j]], add=True)
            return carry

        lax.fori_loop(0, IB, body, 0)
    plsc.subcore_barrier()
    pltpu.sync_copy(acc.at[pl.ds(s * RPS, RPS)],
                    out.at[c, pl.ds(s * RPS, RPS)])


_deg_call = pl.kernel(
    _deg_body,
    out_type=jax.ShapeDtypeStruct((NC, NP, DH), jnp.float32),
    mesh=_MESH,
    scratch_types=[
        pltpu.VMEM_SHARED((NP, DH), jnp.float32),
        pltpu.VMEM((K, DH), jnp.float32),
        pltpu.VMEM((IB, K), jnp.int32),
    ],
)


def _msg_body(hs, srcm, dstm, zeros_hbm, out, acc, src_v, dst_v,
              rows0, rows1, rows2, rows3, sem0, sem1, sem2, sem3):
    c = lax.axis_index("c")
    s = lax.axis_index("s")
    wid = s * NC + c
    for r in range(RPS // ZR):
        pltpu.sync_copy(zeros_hbm, acc.at[pl.ds(s * RPS + r * ZR, ZR)])
    plsc.subcore_barrier()

    bufs = (rows0, rows1, rows2, rows3)
    sems = (sem0, sem1, sem2, sem3)

    def gather(j, b):
        pltpu.async_copy(hs.at[src_v.at[j]], bufs[b], sems[b])

    def wait_scat(j, b):
        pltpu.make_async_copy(hs.at[src_v.at[j]], bufs[b], sems[b]).wait()
        pltpu.sync_copy(bufs[b], acc.at[dst_v.at[j]], add=True)

    for blk in range(NB):
        pltpu.sync_copy(srcm.at[wid, blk], src_v)
        pltpu.sync_copy(dstm.at[wid, blk], dst_v)
        gather(0, 0)
        gather(1, 1)
        gather(2, 2)

        def quad(p, carry):
            j = 4 * p
            gather(j + 3, 3)
            wait_scat(j, 0)
            gather(j + 4, 0)
            wait_scat(j + 1, 1)
            gather(j + 5, 1)
            wait_scat(j + 2, 2)
            gather(j + 6, 2)
            wait_scat(j + 3, 3)
            return carry

        lax.fori_loop(0, (IB - 5) // 4, quad, 0)
        gather(IB - 2, 3)
        wait_scat(IB - 5, 0)
        gather(IB - 1, 0)
        wait_scat(IB - 4, 1)
        wait_scat(IB - 3, 2)
        wait_scat(IB - 2, 3)
        wait_scat(IB - 1, 0)
    plsc.subcore_barrier()
    pltpu.sync_copy(acc.at[pl.ds(s * RPS, RPS)],
                    out.at[c, pl.ds(s * RPS, RPS)])


_msg_call = pl.kernel(
    _msg_body,
    out_type=jax.ShapeDtypeStruct((NC, NP, DH), jnp.float32),
    mesh=_MESH,
    scratch_types=[
        pltpu.VMEM_SHARED((NP, DH), jnp.float32),
        pltpu.VMEM((IB, K), jnp.int32),
        pltpu.VMEM((IB, K), jnp.int32),
        pltpu.VMEM((K, DH), jnp.float32),
        pltpu.VMEM((K, DH), jnp.float32),
        pltpu.VMEM((K, DH), jnp.float32),
        pltpu.VMEM((K, DH), jnp.float32),
        pltpu.SemaphoreType.DMA,
        pltpu.SemaphoreType.DMA,
        pltpu.SemaphoreType.DMA,
        pltpu.SemaphoreType.DMA,
    ],
)


_RB = 400


def _deg_to_dis(d0_ref, d1_ref):
    return lax.rsqrt(d0_ref[0] + d1_ref[0] + 1.0)


def _mm1_body(x_ref, w_ref, d0_ref, d1_ref, o_ref):
    dis = _deg_to_dis(d0_ref, d1_ref)
    o_ref[...] = jnp.dot(x_ref[...], w_ref[...],
                         preferred_element_type=jnp.float32) * dis


def _layer2_body(p0_ref, p1_ref, hs1_ref, d0_ref, d1_ref, b1_ref, w2_ref,
                 o_ref):
    dis = _deg_to_dis(d0_ref, d1_ref)
    agg = (p0_ref[0] + p1_ref[0] + hs1_ref[...]) * dis + b1_ref[...]
    h = jnp.maximum(agg, 0.0)
    o_ref[...] = jnp.dot(h, w2_ref[...],
                         preferred_element_type=jnp.float32) * dis


def _pool_body(p0_ref, p1_ref, hs2_ref, d0_ref, d1_ref, b2_ref, batch_ref,
               o_ref, sums, cnts):
    i = pl.program_id(0)

    @pl.when(i == 0)
    def _init():
        sums[...] = jnp.zeros_like(sums)
        cnts[...] = jnp.zeros_like(cnts)

    dis = _deg_to_dis(d0_ref, d1_ref)
    p2 = (p0_ref[0] + p1_ref[0] + hs2_ref[...]) * dis + b2_ref[...]
    gids = lax.broadcasted_iota(jnp.int32, (_RB, G), 1)
    onehot = (batch_ref[...] == gids).astype(jnp.float32)
    sums[...] += lax.dot_general(onehot, p2, (((0,), (0,)), ((), ())),
                                 preferred_element_type=jnp.float32)
    cnts[...] += lax.dot_general(onehot, jnp.ones((_RB, 1), jnp.float32),
                                 (((0,), (0,)), ((), ())),
                                 preferred_element_type=jnp.float32)

    @pl.when(i == pl.num_programs(0) - 1)
    def _fin():
        o_ref[...] = sums[...] / jnp.maximum(cnts[...], 1.0)


def _row_spec(w):
    return pl.BlockSpec((_RB, w), lambda i: (i, 0))


def _part_spec(part, w):
    return pl.BlockSpec((1, _RB, w), lambda i, _p=part: (_p, i, 0))


def _full_spec(shape):
    return pl.BlockSpec(shape, lambda i: (0,) * len(shape))


_mm1_call = pl.pallas_call(
    _mm1_body,
    grid=(N // _RB,),
    in_specs=[_row_spec(768), _full_spec((768, DH)),
              _part_spec(0, 1), _part_spec(1, 1)],
    out_specs=_row_spec(DH),
    out_shape=jax.ShapeDtypeStruct((N, DH), jnp.float32),
)

_layer2_call = pl.pallas_call(
    _layer2_body,
    grid=(N // _RB,),
    in_specs=[_part_spec(0, DH), _part_spec(1, DH), _row_spec(DH),
              _part_spec(0, 1), _part_spec(1, 1),
              _full_spec((1, DH)), _full_spec((DH, DH))],
    out_specs=_row_spec(DH),
    out_shape=jax.ShapeDtypeStruct((N, DH), jnp.float32),
)

_pool_call = pl.pallas_call(
    _pool_body,
    grid=(N // _RB,),
    in_specs=[_part_spec(0, DH), _part_spec(1, DH), _row_spec(DH),
              _part_spec(0, 1), _part_spec(1, 1),
              _full_spec((1, DH)), _row_spec(1)],
    out_specs=_full_spec((G, DH)),
    out_shape=jax.ShapeDtypeStruct((G, DH), jnp.float32),
    scratch_shapes=[pltpu.VMEM((G, DH), jnp.float32),
                    pltpu.VMEM((G, 1), jnp.float32)],
)


def kernel(x, edge_index, batch, W1, b1, W2, b2):
    srcm = edge_index[0].astype(jnp.int32).reshape(NW, NB, IB, K)
    dstm = edge_index[1].astype(jnp.int32).reshape(NW, NB, IB, K)
    ones128 = jnp.ones((K, DH), jnp.float32)
    zeros128 = jnp.zeros((ZR, DH), jnp.float32)
    batch2 = batch.astype(jnp.int32).reshape(N, 1)
    b1r = b1.reshape(1, DH)
    b2r = b2.reshape(1, DH)

    degp = _deg_call(dstm, ones128, zeros128)
    degc = degp[:, :, 0:1]
    hs1 = _mm1_call(x, W1, degc, degc)
    m1 = _msg_call(hs1, srcm, dstm, zeros128)
    hs2 = _layer2_call(m1, m1, hs1, degc, degc, b1r, W2)
    m2 = _msg_call(hs2, srcm, dstm, zeros128)
    return _pool_call(m2, m2, hs2, degc, degc, b2r, batch2)


# speedup vs baseline: 21.4709x; 1.0071x over previous
import functools

import jax
import jax.numpy as jnp
from jax import lax
from jax.experimental import pallas as pl
from jax.experimental.pallas import tpu as pltpu
from jax.experimental.pallas import tpu_sc as plsc

N = 10000
E = 320000
G = 64
DH = 128

NC = 2
NS = 16
NW = NC * NS
EPW = E // NW
K = 80
CH = EPW // K
NP = 10240
RPS = NP // NS
ZR = 128
IB = 25
NB = CH // IB

_MESH = plsc.VectorSubcoreMesh(
    core_axis_name="c", subcore_axis_name="s", num_cores=NC, num_subcores=NS)


def _deg_body(dstm, ones_hbm, zeros_hbm, out, acc, ones_v, idx_v):
    c = lax.axis_index("c")
    s = lax.axis_index("s")
    wid = s * NC + c
    pltpu.sync_copy(ones_hbm, ones_v)
    for r in range(RPS // ZR):
        pltpu.sync_copy(zeros_hbm, acc.at[pl.ds(s * RPS + r * ZR, ZR)])
    plsc.subcore_barrier()

    for b in range(NB):
        pltpu.sync_copy(dstm.at[wid, b], idx_v)

        def body(j, carry):
            pltpu.sync_copy(ones_v, acc.at[idx_v.at[j]], add=True)
            return carry

        lax.fori_loop(0, IB, body, 0)
    plsc.subcore_barrier()
    pltpu.sync_copy(acc.at[pl.ds(s * RPS, RPS)],
                    out.at[c, pl.ds(s * RPS, RPS)])


_deg_call = pl.kernel(
    _deg_body,
    out_type=jax.ShapeDtypeStruct((NC, NP, DH), jnp.float32),
    mesh=_MESH,
    scratch_types=[
        pltpu.VMEM_SHARED((NP, DH), jnp.float32),
        pltpu.VMEM((K, DH), jnp.float32),
        pltpu.VMEM((IB, K), jnp.int32),
    ],
)


def _msg_body(hs, srcm, dstm, zeros_hbm, out, acc, src_v, dst_v,
              rows0, rows1, rows2, rows3, sem0, sem1, sem2, sem3):
    c = lax.axis_index("c")
    s = lax.axis_index("s")
    wid = s * NC + c
    for r in range(RPS // ZR):
        pltpu.sync_copy(zeros_hbm, acc.at[pl.ds(s * RPS + r * ZR, ZR)])
    plsc.subcore_barrier()

    bufs = (rows0, rows1, rows2, rows3)
    sems = (sem0, sem1, sem2, sem3)

    def gather(j, b):
        pltpu.async_copy(hs.at[src_v.at[j]], bufs[b], sems[b])

    def wait_scat(j, b):
        pltpu.make_async_copy(hs.at[src_v.at[j]], bufs[b], sems[b]).wait()
        pltpu.sync_copy(bufs[b], acc.at[dst_v.at[j]], add=True)

    for blk in range(NB):
        pltpu.sync_copy(srcm.at[wid, blk], src_v)
        pltpu.sync_copy(dstm.at[wid, blk], dst_v)
        gather(0, 0)
        gather(1, 1)
        gather(2, 2)

        def quad(p, carry):
            j = 4 * p
            gather(j + 3, 3)
            wait_scat(j, 0)
            gather(j + 4, 0)
            wait_scat(j + 1, 1)
            gather(j + 5, 1)
            wait_scat(j + 2, 2)
            gather(j + 6, 2)
            wait_scat(j + 3, 3)
            return carry

        lax.fori_loop(0, (IB - 5) // 4, quad, 0)
        gather(IB - 2, 3)
        wait_scat(IB - 5, 0)
        gather(IB - 1, 0)
        wait_scat(IB - 4, 1)
        wait_scat(IB - 3, 2)
        wait_scat(IB - 2, 3)
        wait_scat(IB - 1, 0)
    plsc.subcore_barrier()
    pltpu.sync_copy(acc.at[pl.ds(s * RPS, RPS)],
                    out.at[c, pl.ds(s * RPS, RPS)])


_msg_call = pl.kernel(
    _msg_body,
    out_type=jax.ShapeDtypeStruct((NC, NP, DH), jnp.float32),
    mesh=_MESH,
    scratch_types=[
        pltpu.VMEM_SHARED((NP, DH), jnp.float32),
        pltpu.VMEM((IB, K), jnp.int32),
        pltpu.VMEM((IB, K), jnp.int32),
        pltpu.VMEM((K, DH), jnp.float32),
        pltpu.VMEM((K, DH), jnp.float32),
        pltpu.VMEM((K, DH), jnp.float32),
        pltpu.VMEM((K, DH), jnp.float32),
        pltpu.SemaphoreType.DMA,
        pltpu.SemaphoreType.DMA,
        pltpu.SemaphoreType.DMA,
        pltpu.SemaphoreType.DMA,
    ],
)


_RB = 400


def _deg_to_dis(d0_ref, d1_ref):
    return lax.rsqrt(d0_ref[0] + d1_ref[0] + 1.0)


def _mm1_body(x_ref, w_ref, o_ref):
    o_ref[...] = jnp.dot(x_ref[...], w_ref[...],
                         preferred_element_type=jnp.float32)


def _scale_body(h_ref, d0_ref, d1_ref, o_ref):
    o_ref[...] = h_ref[...] * _deg_to_dis(d0_ref, d1_ref)


def _layer2_body(p0_ref, p1_ref, hs1_ref, d0_ref, d1_ref, b1_ref, w2_ref,
                 o_ref):
    dis = _deg_to_dis(d0_ref, d1_ref)
    agg = (p0_ref[0] + p1_ref[0] + hs1_ref[...]) * dis + b1_ref[...]
    h = jnp.maximum(agg, 0.0)
    o_ref[...] = jnp.dot(h, w2_ref[...],
                         preferred_element_type=jnp.float32) * dis


def _pool_body(p0_ref, p1_ref, hs2_ref, d0_ref, d1_ref, b2_ref, batch_ref,
               o_ref, sums, cnts):
    i = pl.program_id(0)

    @pl.when(i == 0)
    def _init():
        sums[...] = jnp.zeros_like(sums)
        cnts[...] = jnp.zeros_like(cnts)

    dis = _deg_to_dis(d0_ref, d1_ref)
    p2 = (p0_ref[0] + p1_ref[0] + hs2_ref[...]) * dis + b2_ref[...]
    gids = lax.broadcasted_iota(jnp.int32, (_RB, G), 1)
    onehot = (batch_ref[...] == gids).astype(jnp.float32)
    sums[...] += lax.dot_general(onehot, p2, (((0,), (0,)), ((), ())),
                                 preferred_element_type=jnp.float32)
    cnts[...] += lax.dot_general(onehot, jnp.ones((_RB, 1), jnp.float32),
                                 (((0,), (0,)), ((), ())),
                                 preferred_element_type=jnp.float32)

    @pl.when(i == pl.num_programs(0) - 1)
    def _fin():
        o_ref[...] = sums[...] / jnp.maximum(cnts[...], 1.0)


def _row_spec(w):
    return pl.BlockSpec((_RB, w), lambda i: (i, 0))


def _part_spec(part, w):
    return pl.BlockSpec((1, _RB, w), lambda i, _p=part: (_p, i, 0))


def _full_spec(shape):
    return pl.BlockSpec(shape, lambda i: (0,) * len(shape))


_mm1_call = pl.pallas_call(
    _mm1_body,
    grid=(N // _RB,),
    in_specs=[_row_spec(768), _full_spec((768, DH))],
    out_specs=_row_spec(DH),
    out_shape=jax.ShapeDtypeStruct((N, DH), jnp.float32),
)

_scale_call = pl.pallas_call(
    _scale_body,
    grid=(N // _RB,),
    in_specs=[_row_spec(DH), _part_spec(0, 1), _part_spec(1, 1)],
    out_specs=_row_spec(DH),
    out_shape=jax.ShapeDtypeStruct((N, DH), jnp.float32),
)

_layer2_call = pl.pallas_call(
    _layer2_body,
    grid=(N // _RB,),
    in_specs=[_part_spec(0, DH), _part_spec(1, DH), _row_spec(DH),
              _part_spec(0, 1), _part_spec(1, 1),
              _full_spec((1, DH)), _full_spec((DH, DH))],
    out_specs=_row_spec(DH),
    out_shape=jax.ShapeDtypeStruct((N, DH), jnp.float32),
)

_pool_call = pl.pallas_call(
    _pool_body,
    grid=(N // _RB,),
    in_specs=[_part_spec(0, DH), _part_spec(1, DH), _row_spec(DH),
              _part_spec(0, 1), _part_spec(1, 1),
              _full_spec((1, DH)), _row_spec(1)],
    out_specs=_full_spec((G, DH)),
    out_shape=jax.ShapeDtypeStruct((G, DH), jnp.float32),
    scratch_shapes=[pltpu.VMEM((G, DH), jnp.float32),
                    pltpu.VMEM((G, 1), jnp.float32)],
)


def kernel(x, edge_index, batch, W1, b1, W2, b2):
    srcm = edge_index[0].astype(jnp.int32).reshape(NW, NB, IB, K)
    dstm = edge_index[1].astype(jnp.int32).reshape(NW, NB, IB, K)
    ones128 = jnp.ones((K, DH), jnp.float32)
    zeros128 = jnp.zeros((ZR, DH), jnp.float32)
    batch2 = batch.astype(jnp.int32).reshape(N, 1)
    b1r = b1.reshape(1, DH)
    b2r = b2.reshape(1, DH)

    h1 = _mm1_call(x, W1)
    degp = _deg_call(dstm, ones128, zeros128)
    degc = degp[:, :, 0:1]
    hs1 = _scale_call(h1, degc, degc)
    m1 = _msg_call(hs1, srcm, dstm, zeros128)
    hs2 = _layer2_call(m1, m1, hs1, degc, degc, b1r, W2)
    m2 = _msg_call(hs2, srcm, dstm, zeros128)
    return _pool_call(m2, m2, hs2, degc, degc, b2r, batch2)


# speedup vs baseline: 24.7976x; 1.1549x over previous
import functools

import jax
import jax.numpy as jnp
from jax import lax
from jax.experimental import pallas as pl
from jax.experimental.pallas import tpu as pltpu
from jax.experimental.pallas import tpu_sc as plsc

N = 10000
E = 320000
G = 64
DH = 128

NC = 2
NS = 16
NW = NC * NS
EPW = E // NW
K = 80
CH = EPW // K
NP = 10240
RPS = NP // NS
ZR = 128
IB = 25
NB = CH // IB

_MESH = plsc.VectorSubcoreMesh(
    core_axis_name="c", subcore_axis_name="s", num_cores=NC, num_subcores=NS)


def _deg_body(dstm, ones_hbm, zeros_hbm, out, acc, ones_v, idx_v):
    c = lax.axis_index("c")
    s = lax.axis_index("s")
    wid = s * NC + c
    pltpu.sync_copy(ones_hbm, ones_v)
    pltpu.sync_copy(zeros_hbm, acc.at[pl.ds(s * RPS, RPS)])
    plsc.subcore_barrier()

    for blk in range(NB):
        pltpu.sync_copy(dstm.at[wid, blk], idx_v)

        def body(j, carry):
            pltpu.sync_copy(ones_v, acc.at[idx_v.at[j]], add=True)
            return carry

        lax.fori_loop(0, IB, body, 0)
    plsc.subcore_barrier()
    pltpu.sync_copy(acc.at[pl.ds(s * RPS, RPS)],
                    out.at[c, pl.ds(s * RPS, RPS)])


_deg_call = pl.kernel(
    _deg_body,
    out_type=jax.ShapeDtypeStruct((NC, NP), jnp.float32),
    mesh=_MESH,
    scratch_types=[
        pltpu.VMEM_SHARED((NP,), jnp.float32),
        pltpu.VMEM((K,), jnp.float32),
        pltpu.VMEM((IB, K), jnp.int32),
    ],
)


def _msg_body(hs, srcm, dstm, zeros_hbm, out, acc, src_v, dst_v,
              rows0, rows1, rows2, rows3, sem0, sem1, sem2, sem3):
    c = lax.axis_index("c")
    s = lax.axis_index("s")
    wid = s * NC + c
    for r in range(RPS // ZR):
        pltpu.sync_copy(zeros_hbm, acc.at[pl.ds(s * RPS + r * ZR, ZR)])
    plsc.subcore_barrier()

    bufs = (rows0, rows1, rows2, rows3)
    sems = (sem0, sem1, sem2, sem3)

    def gather(j, b):
        pltpu.async_copy(hs.at[src_v.at[j]], bufs[b], sems[b])

    def wait_scat(j, b):
        pltpu.make_async_copy(hs.at[src_v.at[j]], bufs[b], sems[b]).wait()
        pltpu.sync_copy(bufs[b], acc.at[dst_v.at[j]], add=True)

    for blk in range(NB):
        pltpu.sync_copy(srcm.at[wid, blk], src_v)
        pltpu.sync_copy(dstm.at[wid, blk], dst_v)
        gather(0, 0)
        gather(1, 1)
        gather(2, 2)

        def quad(p, carry):
            j = 4 * p
            gather(j + 3, 3)
            wait_scat(j, 0)
            gather(j + 4, 0)
            wait_scat(j + 1, 1)
            gather(j + 5, 1)
            wait_scat(j + 2, 2)
            gather(j + 6, 2)
            wait_scat(j + 3, 3)
            return carry

        lax.fori_loop(0, (IB - 5) // 4, quad, 0)
        gather(IB - 2, 3)
        wait_scat(IB - 5, 0)
        gather(IB - 1, 0)
        wait_scat(IB - 4, 1)
        wait_scat(IB - 3, 2)
        wait_scat(IB - 2, 3)
        wait_scat(IB - 1, 0)
    plsc.subcore_barrier()
    pltpu.sync_copy(acc.at[pl.ds(s * RPS, RPS)],
                    out.at[c, pl.ds(s * RPS, RPS)])


_msg_call = pl.kernel(
    _msg_body,
    out_type=jax.ShapeDtypeStruct((NC, NP, DH), jnp.float32),
    mesh=_MESH,
    scratch_types=[
        pltpu.VMEM_SHARED((NP, DH), jnp.float32),
        pltpu.VMEM((IB, K), jnp.int32),
        pltpu.VMEM((IB, K), jnp.int32),
        pltpu.VMEM((K, DH), jnp.float32),
        pltpu.VMEM((K, DH), jnp.float32),
        pltpu.VMEM((K, DH), jnp.float32),
        pltpu.VMEM((K, DH), jnp.float32),
        pltpu.SemaphoreType.DMA,
        pltpu.SemaphoreType.DMA,
        pltpu.SemaphoreType.DMA,
        pltpu.SemaphoreType.DMA,
    ],
)


_RB = 400


def _deg_to_dis(d0_ref, d1_ref):
    return lax.rsqrt(d0_ref[0] + d1_ref[0] + 1.0)


def _mm1_body(x_ref, w_ref, o_ref):
    o_ref[...] = jnp.dot(x_ref[...], w_ref[...],
                         preferred_element_type=jnp.float32)


def _scale_body(h_ref, d0_ref, d1_ref, o_ref):
    o_ref[...] = h_ref[...] * _deg_to_dis(d0_ref, d1_ref)


def _layer2_body(p0_ref, p1_ref, hs1_ref, d0_ref, d1_ref, b1_ref, w2_ref,
                 o_ref):
    dis = _deg_to_dis(d0_ref, d1_ref)
    agg = (p0_ref[0] + p1_ref[0] + hs1_ref[...]) * dis + b1_ref[...]
    h = jnp.maximum(agg, 0.0)
    o_ref[...] = jnp.dot(h, w2_ref[...],
                         preferred_element_type=jnp.float32) * dis


def _pool_body(p0_ref, p1_ref, hs2_ref, d0_ref, d1_ref, b2_ref, batch_ref,
               o_ref, sums, cnts):
    i = pl.program_id(0)

    @pl.when(i == 0)
    def _init():
        sums[...] = jnp.zeros_like(sums)
        cnts[...] = jnp.zeros_like(cnts)

    dis = _deg_to_dis(d0_ref, d1_ref)
    p2 = (p0_ref[0] + p1_ref[0] + hs2_ref[...]) * dis + b2_ref[...]
    gids = lax.broadcasted_iota(jnp.int32, (_RB, G), 1)
    onehot = (batch_ref[...] == gids).astype(jnp.float32)
    sums[...] += lax.dot_general(onehot, p2, (((0,), (0,)), ((), ())),
                                 preferred_element_type=jnp.float32)
    cnts[...] += lax.dot_general(onehot, jnp.ones((_RB, 1), jnp.float32),
                                 (((0,), (0,)), ((), ())),
                                 preferred_element_type=jnp.float32)

    @pl.when(i == pl.num_programs(0) - 1)
    def _fin():
        o_ref[...] = sums[...] / jnp.maximum(cnts[...], 1.0)


def _row_spec(w):
    return pl.BlockSpec((_RB, w), lambda i: (i, 0))


def _part_spec(part, w):
    return pl.BlockSpec((1, _RB, w), lambda i, _p=part: (_p, i, 0))


def _full_spec(shape):
    return pl.BlockSpec(shape, lambda i: (0,) * len(shape))


_mm1_call = pl.pallas_call(
    _mm1_body,
    grid=(N // _RB,),
    in_specs=[_row_spec(768), _full_spec((768, DH))],
    out_specs=_row_spec(DH),
    out_shape=jax.ShapeDtypeStruct((N, DH), jnp.float32),
)

_scale_call = pl.pallas_call(
    _scale_body,
    grid=(N // _RB,),
    in_specs=[_row_spec(DH), _part_spec(0, 1), _part_spec(1, 1)],
    out_specs=_row_spec(DH),
    out_shape=jax.ShapeDtypeStruct((N, DH), jnp.float32),
)

_layer2_call = pl.pallas_call(
    _layer2_body,
    grid=(N // _RB,),
    in_specs=[_part_spec(0, DH), _part_spec(1, DH), _row_spec(DH),
              _part_spec(0, 1), _part_spec(1, 1),
              _full_spec((1, DH)), _full_spec((DH, DH))],
    out_specs=_row_spec(DH),
    out_shape=jax.ShapeDtypeStruct((N, DH), jnp.float32),
)

_pool_call = pl.pallas_call(
    _pool_body,
    grid=(N // _RB,),
    in_specs=[_part_spec(0, DH), _part_spec(1, DH), _row_spec(DH),
              _part_spec(0, 1), _part_spec(1, 1),
              _full_spec((1, DH)), _row_spec(1)],
    out_specs=_full_spec((G, DH)),
    out_shape=jax.ShapeDtypeStruct((G, DH), jnp.float32),
    scratch_shapes=[pltpu.VMEM((G, DH), jnp.float32),
                    pltpu.VMEM((G, 1), jnp.float32)],
)


def kernel(x, edge_index, batch, W1, b1, W2, b2):
    srcm = edge_index[0].astype(jnp.int32).reshape(NW, NB, IB, K)
    dstm = edge_index[1].astype(jnp.int32).reshape(NW, NB, IB, K)
    zeros128 = jnp.zeros((ZR, DH), jnp.float32)
    onesk = jnp.ones((K,), jnp.float32)
    zerosk = jnp.zeros((RPS,), jnp.float32)
    batch2 = batch.astype(jnp.int32).reshape(N, 1)
    b1r = b1.reshape(1, DH)
    b2r = b2.reshape(1, DH)

    h1 = _mm1_call(x, W1)
    degc = _deg_call(dstm, onesk, zerosk).reshape(NC, NP, 1)
    hs1 = _scale_call(h1, degc, degc)
    m1 = _msg_call(hs1, srcm, dstm, zeros128)
    hs2 = _layer2_call(m1, m1, hs1, degc, degc, b1r, W2)
    m2 = _msg_call(hs2, srcm, dstm, zeros128)
    return _pool_call(m2, m2, hs2, degc, degc, b2r, batch2)
